# baseline (speedup 1.0000x reference)
import jax
import jax.numpy as jnp
from jax.experimental import pallas as pl

_D = 8732
_DP = 8832
_C = 21
_T = 8
_B = 32
_THRESH = 0.5
_NEGPOS = 3
_POS_INF_BITS = 0x7F800000


def _mbl_body(tgt_ref, dboxT_ref, predT_ref, out_ref):
    b = pl.program_id(0)
    pf = predT_ref[0]

    t = tgt_ref[0]
    tx1 = t[:, 0:1]
    ty1 = t[:, 1:2]
    tx2 = t[:, 2:3]
    ty2 = t[:, 3:4]
    lab = t[:, 4:5]

    pcx = dboxT_ref[0:1, :]
    pcy = dboxT_ref[1:2, :]
    pw = dboxT_ref[2:3, :]
    ph = dboxT_ref[3:4, :]
    px1 = pcx - pw / 2.0
    py1 = pcy - ph / 2.0
    px2 = pcx + pw / 2.0
    py2 = pcy + ph / 2.0

    iw = jnp.maximum(jnp.minimum(tx2, px2) - jnp.maximum(tx1, px1), 0.0)
    ih = jnp.maximum(jnp.minimum(ty2, py2) - jnp.maximum(ty1, py1), 0.0)
    inter = iw * ih
    area_t = (tx2 - tx1) * (ty2 - ty1)
    area_p = (px2 - px1) * (py2 - py1)
    ov = inter / (area_t + area_p - inter)

    lane = jax.lax.broadcasted_iota(jnp.int32, (_T, _DP), 1)
    row = jax.lax.broadcasted_iota(jnp.int32, (_T, _DP), 0)

    rmax = jnp.max(ov, axis=1, keepdims=True)
    bpi = jnp.min(jnp.where(ov == rmax, lane, _D), axis=1, keepdims=True)
    bto = jnp.max(ov, axis=0, keepdims=True)
    bti = jnp.min(jnp.where(ov == bto, row, _T), axis=0, keepdims=True)

    fm = lane == bpi
    forced = jnp.max(jnp.where(fm, row, -1), axis=0, keepdims=True)
    bto = jnp.where(forced >= 0, 2.0, bto)
    bti = jnp.where(forced >= 0, forced, bti)

    sel = (row == bti).astype(jnp.float32)
    labg = jnp.sum(sel * lab, axis=0, keepdims=True)
    mx1 = jnp.sum(sel * tx1, axis=0, keepdims=True)
    my1 = jnp.sum(sel * ty1, axis=0, keepdims=True)
    mx2 = jnp.sum(sel * tx2, axis=0, keepdims=True)
    my2 = jnp.sum(sel * ty2, axis=0, keepdims=True)

    conf_t = jnp.where(bto < _THRESH, 0.0, labg + 1.0)
    pos = conf_t > 0.0
    posf = pos.astype(jnp.float32)

    g_cx = ((mx1 + mx2) / 2.0 - pcx) / (0.1 * pw)
    g_cy = ((my1 + my2) / 2.0 - pcy) / (0.1 * ph)
    g_w = jnp.log((mx2 - mx1) / pw) / 0.2
    g_h = jnp.log((my2 - my1) / ph) / 0.2

    ll = jnp.float32(0.0)
    for i, g in enumerate((g_cx, g_cy, g_w, g_h)):
        d = pf[i:i + 1, :] - g
        ad = jnp.abs(d)
        sl1 = jnp.where(ad < 1.0, 0.5 * d * d, ad - 0.5)
        ll = ll + jnp.sum(sl1 * posf)

    prow = jax.lax.broadcasted_iota(jnp.int32, (25, _DP), 0)
    is_cls = prow >= 4
    neg_inf = jnp.float32(-jnp.inf)
    m = jnp.max(jnp.where(is_cls, pf, neg_inf), axis=0, keepdims=True)
    s = jnp.sum(jnp.where(is_cls, jnp.exp(pf - m), 0.0), axis=0,
                keepdims=True)
    lse = m + jnp.log(s)
    lbl_i = conf_t.astype(jnp.int32)
    logit_l = jnp.sum(jnp.where(prow == lbl_i + 4, pf, 0.0), axis=0,
                      keepdims=True)
    ce = lse - logit_l
    real = jax.lax.broadcasted_iota(jnp.int32, (1, _DP), 1) < _D
    ce = jnp.where(real, ce, 0.0)

    sum_pos_ce = jnp.sum(jnp.where(pos, ce, 0.0))
    npos_i = jnp.sum(pos.astype(jnp.int32))
    k = jnp.minimum(_NEGPOS * npos_i, _D)

    ce_neg = jnp.where(pos, 0.0, ce)
    ce_int = jax.lax.bitcast_convert_type(ce_neg, jnp.int32)

    def srch(_, lo_hi):
        lo, hi = lo_hi
        mid = lo + (hi - lo) // 2
        cnt = jnp.sum((ce_int > mid).astype(jnp.int32))
        take = cnt < k
        return (jnp.where(take, lo, mid + 1), jnp.where(take, mid, hi))

    _, t_int = jax.lax.fori_loop(
        0, 31, srch, (jnp.int32(0), jnp.int32(_POS_INF_BITS)))
    gt = ce_int > t_int
    sum_gt = jnp.sum(jnp.where(gt, ce_neg, 0.0))
    cnt_gt = jnp.sum(gt.astype(jnp.int32))
    t_f = jnp.max(jnp.where(ce_int == t_int, ce_neg, 0.0))
    lc = sum_pos_ce + sum_gt + (k - cnt_gt).astype(jnp.float32) * t_f

    lane_o = jax.lax.broadcasted_iota(jnp.int32, (1, 128), 1)
    vec = (jnp.where(lane_o == 0, ll, 0.0)
           + jnp.where(lane_o == 1, lc, 0.0)
           + jnp.where(lane_o == 2, npos_i.astype(jnp.float32), 0.0))

    @pl.when(b == 0)
    def _():
        out_ref[...] = vec

    @pl.when(b != 0)
    def _():
        out_ref[...] = out_ref[...] + vec


@jax.jit
def kernel(predictions, targets, dbox_list):
    predT = jnp.pad(jnp.transpose(predictions, (0, 2, 1)),
                    ((0, 0), (0, 0), (0, _DP - _D)))
    far = jnp.tile(jnp.array([[-100.0, -100.0, 1.0, 1.0]], jnp.float32),
                   (_DP - _D, 1))
    dboxT = jnp.concatenate([dbox_list, far], axis=0).T
    out = pl.pallas_call(
        _mbl_body,
        grid=(_B,),
        in_specs=[
            pl.BlockSpec((1, _T, 5), lambda b: (b, 0, 0)),
            pl.BlockSpec((4, _DP), lambda b: (0, 0)),
            pl.BlockSpec((1, 25, _DP), lambda b: (b, 0, 0)),
        ],
        out_specs=pl.BlockSpec((1, 128), lambda b: (0, 0)),
        out_shape=jax.ShapeDtypeStruct((1, 128), jnp.float32),
    )(targets, dboxT, predT)
    n = out[0, 2]
    return (out[0, 0] / n, out[0, 1] / n)


# speedup vs baseline: 2.0501x; 2.0501x over previous
import jax
import jax.numpy as jnp
from jax.experimental import pallas as pl
from jax.experimental.pallas import tpu as pltpu

_D = 8732
_DP = _D
_C = 21
_T = 8
_B = 32
_THRESH = 0.5
_NEGPOS = 3
_POS_INF_BITS = 0x7F800000


def _mbl_body(tgt_ref, dboxT_ref, predT_ref, out_ref, ce_ref, k_ref):
    b = pl.program_id(0)
    pf = predT_ref[0]

    t = tgt_ref[0]
    tx1 = t[:, 0:1]
    ty1 = t[:, 1:2]
    tx2 = t[:, 2:3]
    ty2 = t[:, 3:4]
    lab = t[:, 4:5]

    pcx = dboxT_ref[0:1, :]
    pcy = dboxT_ref[1:2, :]
    pw = dboxT_ref[2:3, :]
    ph = dboxT_ref[3:4, :]
    px1 = pcx - pw / 2.0
    py1 = pcy - ph / 2.0
    px2 = pcx + pw / 2.0
    py2 = pcy + ph / 2.0

    iw = jnp.maximum(jnp.minimum(tx2, px2) - jnp.maximum(tx1, px1), 0.0)
    ih = jnp.maximum(jnp.minimum(ty2, py2) - jnp.maximum(ty1, py1), 0.0)
    inter = iw * ih
    area_t = (tx2 - tx1) * (ty2 - ty1)
    area_p = (px2 - px1) * (py2 - py1)
    ov = inter / (area_t + area_p - inter)

    lane = jax.lax.broadcasted_iota(jnp.int32, (_T, _DP), 1)
    row = jax.lax.broadcasted_iota(jnp.int32, (_T, _DP), 0)

    rmax = jnp.max(ov, axis=1, keepdims=True)
    bpi = jnp.min(jnp.where(ov == rmax, lane, _D), axis=1, keepdims=True)
    bto = jnp.max(ov, axis=0, keepdims=True)
    bti = jnp.min(jnp.where(ov == bto, row, _T), axis=0, keepdims=True)

    fm = lane == bpi
    forced = jnp.max(jnp.where(fm, row, -1), axis=0, keepdims=True)
    bto = jnp.where(forced >= 0, 2.0, bto)
    bti = jnp.where(forced >= 0, forced, bti)

    sel = (row == bti).astype(jnp.float32)
    labg = jnp.sum(sel * lab, axis=0, keepdims=True)
    mx1 = jnp.sum(sel * tx1, axis=0, keepdims=True)
    my1 = jnp.sum(sel * ty1, axis=0, keepdims=True)
    mx2 = jnp.sum(sel * tx2, axis=0, keepdims=True)
    my2 = jnp.sum(sel * ty2, axis=0, keepdims=True)

    conf_t = jnp.where(bto < _THRESH, 0.0, labg + 1.0)
    pos = conf_t > 0.0
    posf = pos.astype(jnp.float32)

    g_cx = ((mx1 + mx2) / 2.0 - pcx) / (0.1 * pw)
    g_cy = ((my1 + my2) / 2.0 - pcy) / (0.1 * ph)
    g_w = jnp.log((mx2 - mx1) / pw) / 0.2
    g_h = jnp.log((my2 - my1) / ph) / 0.2

    ll = jnp.float32(0.0)
    for i, g in enumerate((g_cx, g_cy, g_w, g_h)):
        d = pf[i:i + 1, :] - g
        ad = jnp.abs(d)
        sl1 = jnp.where(ad < 1.0, 0.5 * d * d, ad - 0.5)
        ll = ll + jnp.sum(sl1 * posf)

    prow = jax.lax.broadcasted_iota(jnp.int32, (25, _DP), 0)
    is_cls = prow >= 4
    neg_inf = jnp.float32(-jnp.inf)
    m = jnp.max(jnp.where(is_cls, pf, neg_inf), axis=0, keepdims=True)
    s = jnp.sum(jnp.where(is_cls, jnp.exp(pf - m), 0.0), axis=0,
                keepdims=True)
    lse = m + jnp.log(s)
    lbl_i = conf_t.astype(jnp.int32)
    logit_l = jnp.sum(jnp.where(prow == lbl_i + 4, pf, 0.0), axis=0,
                      keepdims=True)
    ce = lse - logit_l

    sum_pos_ce = jnp.sum(jnp.where(pos, ce, 0.0))
    npos_i = jnp.sum(pos.astype(jnp.int32))
    k = jnp.minimum(_NEGPOS * npos_i, _D)

    ce_ref[pl.ds(b, 1), :] = jnp.where(pos, 0.0, ce)
    lane_o = jax.lax.broadcasted_iota(jnp.int32, (1, 128), 1)
    k_ref[pl.ds(b, 1), :] = jnp.where(lane_o == 0, k, 0)

    vec = (jnp.where(lane_o == 0, ll, 0.0)
           + jnp.where(lane_o == 1, sum_pos_ce, 0.0)
           + jnp.where(lane_o == 2, npos_i.astype(jnp.float32), 0.0))

    @pl.when(b == 0)
    def _():
        out_ref[...] = vec

    @pl.when(b != 0)
    def _():
        out_ref[...] = out_ref[...] + vec

    @pl.when(b == _B - 1)
    def _():
        ce_all = ce_ref[...]
        ce_int = jax.lax.bitcast_convert_type(ce_all, jnp.int32)
        kcol = k_ref[:, 0:1]

        def srch(_, lo_hi):
            lo, hi = lo_hi
            mid = lo + (hi - lo) // 2
            cnt = jnp.sum((ce_int > mid).astype(jnp.int32), axis=1,
                          keepdims=True)
            take = cnt < kcol
            return (jnp.where(take, lo, mid + 1), jnp.where(take, mid, hi))

        zeros = jnp.zeros((_B, 1), jnp.int32)
        _, t_int = jax.lax.fori_loop(
            0, 31, srch, (zeros, jnp.full((_B, 1), _POS_INF_BITS, jnp.int32)))
        gt = ce_int > t_int
        sum_gt = jnp.sum(jnp.where(gt, ce_all, 0.0), axis=1, keepdims=True)
        cnt_gt = jnp.sum(gt.astype(jnp.int32), axis=1, keepdims=True)
        t_f = jnp.max(jnp.where(ce_int == t_int, ce_all, 0.0), axis=1,
                      keepdims=True)
        tk = sum_gt + (kcol - cnt_gt).astype(jnp.float32) * t_f
        total_tk = jnp.sum(tk)
        out_ref[...] = out_ref[...] + jnp.where(lane_o == 1, total_tk, 0.0)


@jax.jit
def kernel(predictions, targets, dbox_list):
    predT = jnp.transpose(predictions, (0, 2, 1))
    dboxT = dbox_list.T
    out = pl.pallas_call(
        _mbl_body,
        grid=(_B,),
        in_specs=[
            pl.BlockSpec((1, _T, 5), lambda b: (b, 0, 0)),
            pl.BlockSpec((4, _DP), lambda b: (0, 0)),
            pl.BlockSpec((1, 25, _DP), lambda b: (b, 0, 0)),
        ],
        out_specs=pl.BlockSpec((1, 128), lambda b: (0, 0)),
        out_shape=jax.ShapeDtypeStruct((1, 128), jnp.float32),
        scratch_shapes=[
            pltpu.VMEM((_B, _DP), jnp.float32),
            pltpu.VMEM((_B, 128), jnp.int32),
        ],
    )(targets, dboxT, predT)
    n = out[0, 2]
    return (out[0, 0] / n, out[0, 1] / n)


# speedup vs baseline: 2.3657x; 1.1540x over previous
import jax
import jax.numpy as jnp
from jax.experimental import pallas as pl
from jax.experimental.pallas import tpu as pltpu

_D = 8732
_DP = _D
_C = 21
_T = 8
_B = 32
_THRESH = 0.5
_NEGPOS = 3
_POS_INF_BITS = 0x7F800000


def _mbl_body(tgt_ref, dboxT_ref, predT_ref, out_ref, ce_ref, pos_ref,
              acc_ref):
    b = pl.program_id(0)
    pf = predT_ref[0]

    t = tgt_ref[0]
    tx1 = t[:, 0:1]
    ty1 = t[:, 1:2]
    tx2 = t[:, 2:3]
    ty2 = t[:, 3:4]
    lab = t[:, 4:5]

    pcx = dboxT_ref[0:1, :]
    pcy = dboxT_ref[1:2, :]
    pw = dboxT_ref[2:3, :]
    ph = dboxT_ref[3:4, :]
    px1 = pcx - pw / 2.0
    py1 = pcy - ph / 2.0
    px2 = pcx + pw / 2.0
    py2 = pcy + ph / 2.0

    iw = jnp.maximum(jnp.minimum(tx2, px2) - jnp.maximum(tx1, px1), 0.0)
    ih = jnp.maximum(jnp.minimum(ty2, py2) - jnp.maximum(ty1, py1), 0.0)
    inter = iw * ih
    area_t = (tx2 - tx1) * (ty2 - ty1)
    area_p = (px2 - px1) * (py2 - py1)
    ov = inter / (area_t + area_p - inter)

    lane = jax.lax.broadcasted_iota(jnp.int32, (_T, _DP), 1)
    row = jax.lax.broadcasted_iota(jnp.int32, (_T, _DP), 0)

    rmax = jnp.max(ov, axis=1, keepdims=True)
    bpi = jnp.min(jnp.where(ov == rmax, lane, _D), axis=1, keepdims=True)
    bto = jnp.max(ov, axis=0, keepdims=True)
    bti = jnp.min(jnp.where(ov == bto, row, _T), axis=0, keepdims=True)

    fm = lane == bpi
    forced = jnp.max(jnp.where(fm, row, -1), axis=0, keepdims=True)
    bto = jnp.where(forced >= 0, 2.0, bto)
    bti = jnp.where(forced >= 0, forced, bti)

    sel = (row == bti).astype(jnp.float32)
    dn = (((0,), (0,)), ((), ()))

    def gath(col):
        return jax.lax.dot_general(col, sel, dn,
                                   preferred_element_type=jnp.float32)

    labg = gath(lab)
    mx1 = gath(tx1)
    my1 = gath(ty1)
    mx2 = gath(tx2)
    my2 = gath(ty2)

    conf_t = jnp.where(bto < _THRESH, 0.0, labg + 1.0)
    pos = conf_t > 0.0
    posf = pos.astype(jnp.float32)

    g_cx = ((mx1 + mx2) / 2.0 - pcx) / (0.1 * pw)
    g_cy = ((my1 + my2) / 2.0 - pcy) / (0.1 * ph)
    g_w = jnp.log((mx2 - mx1) / pw) / 0.2
    g_h = jnp.log((my2 - my1) / ph) / 0.2

    llv = jnp.zeros((1, _DP), jnp.float32)
    for i, g in enumerate((g_cx, g_cy, g_w, g_h)):
        d = pf[i:i + 1, :] - g
        ad = jnp.abs(d)
        sl1 = jnp.where(ad < 1.0, 0.5 * d * d, ad - 0.5)
        llv = llv + sl1 * posf

    prow = jax.lax.broadcasted_iota(jnp.int32, (25, _DP), 0)
    is_cls = prow >= 4
    neg_inf = jnp.float32(-jnp.inf)
    m = jnp.max(jnp.where(is_cls, pf, neg_inf), axis=0, keepdims=True)
    s = jnp.sum(jnp.where(is_cls, jnp.exp(pf - m), 0.0), axis=0,
                keepdims=True)
    lse = m + jnp.log(s)
    lbl_i = conf_t.astype(jnp.int32)
    logit_l = jnp.sum(jnp.where(prow == lbl_i + 4, pf, 0.0), axis=0,
                      keepdims=True)
    ce = lse - logit_l

    ce_ref[pl.ds(b, 1), :] = jnp.where(pos, 0.0, ce)
    pos_ref[pl.ds(b, 1), :] = posf
    pcev = jnp.where(pos, ce, 0.0)

    @pl.when(b == 0)
    def _():
        acc_ref[0:1, :] = llv
        acc_ref[1:2, :] = pcev

    @pl.when(b != 0)
    def _():
        acc_ref[0:1, :] = acc_ref[0:1, :] + llv
        acc_ref[1:2, :] = acc_ref[1:2, :] + pcev

    @pl.when(b == _B - 1)
    def _():
        ce_all = ce_ref[...]
        ce_int = jax.lax.bitcast_convert_type(ce_all, jnp.int32)
        npos_col = jnp.sum(pos_ref[...], axis=1, keepdims=True)
        kcol = jnp.minimum(_NEGPOS * npos_col.astype(jnp.int32), _D)

        def srch(_, lo_hi):
            lo, hi = lo_hi
            mid = lo + (hi - lo) // 2
            cnt = jnp.sum((ce_int > mid).astype(jnp.int32), axis=1,
                          keepdims=True)
            take = cnt < kcol
            return (jnp.where(take, lo, mid + 1), jnp.where(take, mid, hi))

        zeros = jnp.zeros((_B, 1), jnp.int32)
        _, t_int = jax.lax.fori_loop(
            0, 31, srch, (zeros, jnp.full((_B, 1), _POS_INF_BITS, jnp.int32)))
        gt = ce_int > t_int
        sum_gt = jnp.sum(jnp.where(gt, ce_all, 0.0), axis=1, keepdims=True)
        cnt_gt = jnp.sum(gt.astype(jnp.int32), axis=1, keepdims=True)
        t_f = jnp.max(jnp.where(ce_int == t_int, ce_all, 0.0), axis=1,
                      keepdims=True)
        tk = sum_gt + (kcol - cnt_gt).astype(jnp.float32) * t_f
        lc = jnp.sum(tk) + jnp.sum(acc_ref[1:2, :])
        ll = jnp.sum(acc_ref[0:1, :])
        n_tot = jnp.sum(npos_col)
        lane_o = jax.lax.broadcasted_iota(jnp.int32, (1, 128), 1)
        out_ref[...] = (jnp.where(lane_o == 0, ll, 0.0)
                        + jnp.where(lane_o == 1, lc, 0.0)
                        + jnp.where(lane_o == 2, n_tot, 0.0))


@jax.jit
def kernel(predictions, targets, dbox_list):
    predT = jnp.transpose(predictions, (0, 2, 1))
    dboxT = dbox_list.T
    out = pl.pallas_call(
        _mbl_body,
        grid=(_B,),
        in_specs=[
            pl.BlockSpec((1, _T, 5), lambda b: (b, 0, 0)),
            pl.BlockSpec((4, _DP), lambda b: (0, 0)),
            pl.BlockSpec((1, 25, _DP), lambda b: (b, 0, 0)),
        ],
        out_specs=pl.BlockSpec((1, 128), lambda b: (0, 0)),
        out_shape=jax.ShapeDtypeStruct((1, 128), jnp.float32),
        scratch_shapes=[
            pltpu.VMEM((_B, _DP), jnp.float32),
            pltpu.VMEM((_B, _DP), jnp.float32),
            pltpu.VMEM((2, _DP), jnp.float32),
        ],
    )(targets, dboxT, predT)
    n = out[0, 2]
    return (out[0, 0] / n, out[0, 1] / n)
